# baseline (speedup 1.0000x reference)
import functools

import jax
import jax.numpy as jnp
from jax import lax
from jax.experimental import pallas as pl
from jax.experimental.pallas import tpu as pltpu
from jax.experimental.pallas import tpu_sc as plsc

BATCH = 4096
SEQ = 200
DIM = 64
PDIM = 128

NC = 2
NS = 16
NW = NC * NS
ROWS_PER_W = BATCH // NW
N_PASS = 2
ROWS_PER_P = ROWS_PER_W // N_PASS
IDX_PER_W = ROWS_PER_W * SEQ
IDX_PER_P = ROWS_PER_P * SEQ
SUB = 128
SUBS_PER_CHUNK = 2
CHUNK = SUB * SUBS_PER_CHUNK
N_CHUNKS = IDX_PER_P // CHUNK
IDX_ROWS_PER_W = IDX_PER_W // SUB
IDX_ROWS_PER_P = IDX_PER_P // SUB


def _sc_pool(ids2, dest, zeros, table_p):
    mesh = plsc.VectorSubcoreMesh(core_axis_name="c", subcore_axis_name="s")

    @functools.partial(
        pl.kernel,
        mesh=mesh,
        out_type=jax.ShapeDtypeStruct((BATCH, DIM), jnp.float32),
        compiler_params=pltpu.CompilerParams(use_tc_tiling_on_sc=False),
        scratch_types=[
            pltpu.VMEM((IDX_ROWS_PER_W, SUB), jnp.int32),
            pltpu.VMEM((IDX_ROWS_PER_P + 1, SUB), jnp.int32),
            pltpu.VMEM((CHUNK, PDIM), jnp.float32),
            pltpu.VMEM((CHUNK, PDIM), jnp.float32),
            pltpu.VMEM_SHARED((NS * ROWS_PER_P, PDIM), jnp.float32),
            pltpu.SemaphoreType.DMA,
            pltpu.SemaphoreType.DMA,
            pltpu.SemaphoreType.DMA,
        ],
    )
    def k(ids_hbm, dest_hbm, zeros_hbm, table_hbm, out_hbm,
          idx_all, dest_all, rows0, rows1, acc, gsem0, gsem1, ssem):
        cid = lax.axis_index("c")
        sid = lax.axis_index("s")
        wid = cid * NS + sid
        idx_base = wid * IDX_ROWS_PER_W

        pltpu.sync_copy(ids_hbm.at[pl.ds(idx_base, IDX_ROWS_PER_W)], idx_all)
        pltpu.sync_copy(dest_hbm, dest_all)
        off = (sid * ROWS_PER_P).astype(jnp.int32)

        def add_off(r, carry):
            for c4 in range(SUB // 16):
                sl = pl.ds(c4 * 16, 16)
                dest_all[r, sl] = dest_all[r, sl] + off
            return carry

        lax.fori_loop(0, IDX_ROWS_PER_P + 1, add_off, 0)

        def run_pass(p):
            irow0 = p * IDX_ROWS_PER_P

            pltpu.sync_copy(zeros_hbm,
                            acc.at[pl.ds(sid * ROWS_PER_P, ROWS_PER_P)])

            def start_gather(c, buf, sem):
                for s in range(SUBS_PER_CHUNK):
                    pltpu.async_copy(
                        table_hbm.at[idx_all.at[irow0 + c * SUBS_PER_CHUNK + s]],
                        buf.at[pl.ds(s * SUB, SUB)],
                        sem,
                    )

            def wait_gather(buf, sem):
                for s in range(SUBS_PER_CHUNK):
                    pltpu.make_async_copy(
                        table_hbm.at[idx_all.at[s]],
                        buf.at[pl.ds(s * SUB, SUB)],
                        sem,
                    ).wait()

            def scatter_chunk(c, buf):
                for s in range(SUBS_PER_CHUNK):
                    pltpu.async_copy(
                        buf.at[pl.ds(s * SUB, SUB)],
                        acc.at[dest_all.at[c * SUBS_PER_CHUNK + s]],
                        ssem,
                        add=True,
                    ).wait()

            def process(c, buf, sem, nxt_buf, nxt_sem):
                @pl.when(c + 1 < N_CHUNKS)
                def _():
                    start_gather(c + 1, nxt_buf, nxt_sem)
                wait_gather(buf, sem)
                scatter_chunk(c, buf)

            start_gather(0, rows0, gsem0)

            def body(i, carry):
                c = i * 2
                process(c, rows0, gsem0, rows1, gsem1)
                process(c + 1, rows1, gsem1, rows0, gsem0)
                return carry

            lax.fori_loop(0, N_CHUNKS // 2, body, 0)

            pltpu.async_copy(
                acc.at[dest_all.at[IDX_ROWS_PER_P]],
                rows0.at[pl.ds(0, SUB)],
                gsem0,
            ).wait()
            pltpu.sync_copy(
                rows0.at[pl.ds(0, ROWS_PER_P), pl.ds(0, DIM)],
                out_hbm.at[pl.ds(wid * ROWS_PER_W + p * ROWS_PER_P,
                                 ROWS_PER_P)])

        for p in range(N_PASS):
            run_pass(p)

    return k(ids2, dest, zeros, table_p)


def _norm_body(s_ref, o_ref):
    x = s_ref[...] * (1.0 / SEQ)
    ss = jnp.sum(x * x, axis=1, keepdims=True)
    n = jnp.sqrt(ss)
    o_ref[...] = x / jnp.maximum(n, 1e-12)


def _normalize(sums):
    return pl.pallas_call(
        _norm_body,
        out_shape=jax.ShapeDtypeStruct((BATCH, DIM), jnp.float32),
    )(sums)


def kernel(input_ids, table):
    table_p = jnp.pad(table, ((0, 0), (0, PDIM - DIM)))
    ids2 = input_ids.astype(jnp.int32).reshape(-1, SUB)
    base = ((jnp.arange(IDX_PER_P, dtype=jnp.int32) // SEQ)
            % ROWS_PER_P).reshape(IDX_ROWS_PER_P, SUB)
    ident = (jnp.arange(SUB, dtype=jnp.int32) % ROWS_PER_P)[None, :]
    dest = jnp.concatenate([base, ident], axis=0)
    zeros = jnp.zeros((ROWS_PER_P, PDIM), jnp.float32)
    sums = _sc_pool(ids2, dest, zeros, table_p)
    return _normalize(sums)


# speedup vs baseline: 1.1728x; 1.1728x over previous
import functools

import jax
import jax.numpy as jnp
from jax import lax
from jax.experimental import pallas as pl
from jax.experimental.pallas import tpu as pltpu
from jax.experimental.pallas import tpu_sc as plsc

BATCH = 4096
SEQ = 200
DIM = 64
PDIM = 128
VOCAB2 = 2 * 1000000

NC = 2
NS = 16
NW = NC * NS
ROWS_PER_W = BATCH // NW
IDX_PER_W = ROWS_PER_W * SEQ
SUB = 128
SUBS_PER_CHUNK = 2
CHUNK = SUB * SUBS_PER_CHUNK
N_CHUNKS = IDX_PER_W // CHUNK
IDX_ROWS_PER_W = IDX_PER_W // SUB


def _sc_pool(ids2, dest, zeros, table2):
    mesh = plsc.VectorSubcoreMesh(core_axis_name="c", subcore_axis_name="s")

    @functools.partial(
        pl.kernel,
        mesh=mesh,
        out_type=jax.ShapeDtypeStruct((BATCH, DIM), jnp.float32),
        compiler_params=pltpu.CompilerParams(use_tc_tiling_on_sc=False),
        scratch_types=[
            pltpu.VMEM((IDX_ROWS_PER_W, SUB), jnp.int32),
            pltpu.VMEM((IDX_ROWS_PER_W + 1, SUB), jnp.int32),
            pltpu.VMEM((CHUNK, DIM), jnp.float32),
            pltpu.VMEM((CHUNK, DIM), jnp.float32),
            pltpu.VMEM((SUB, DIM), jnp.float32),
            pltpu.VMEM_SHARED((NS * ROWS_PER_W, DIM), jnp.float32),
            pltpu.SemaphoreType.DMA,
            pltpu.SemaphoreType.DMA,
            pltpu.SemaphoreType.DMA,
        ],
    )
    def k(ids_hbm, dest_hbm, zeros_hbm, table_hbm, out_hbm,
          idx_all, dest_all, rows0, rows1, zbuf, acc, gsem0, gsem1, ssem):
        cid = lax.axis_index("c")
        sid = lax.axis_index("s")
        wid = cid * NS + sid
        idx_base = wid * IDX_ROWS_PER_W

        pltpu.sync_copy(ids_hbm.at[pl.ds(idx_base, IDX_ROWS_PER_W)], idx_all)
        pltpu.sync_copy(dest_hbm, dest_all)
        pltpu.sync_copy(zeros_hbm, zbuf)
        off = (sid * ROWS_PER_W).astype(jnp.int32)

        def add_off(r, carry):
            for c4 in range(SUB // 16):
                sl = pl.ds(c4 * 16, 16)
                dest_all[r, sl] = dest_all[r, sl] + off
            return carry

        lax.fori_loop(0, IDX_ROWS_PER_W + 1, add_off, 0)

        pltpu.async_copy(zbuf, acc.at[dest_all.at[IDX_ROWS_PER_W]],
                         ssem).wait()

        def start_gather(c, buf, sem):
            for s in range(SUBS_PER_CHUNK):
                pltpu.async_copy(
                    table_hbm.at[idx_all.at[c * SUBS_PER_CHUNK + s]],
                    buf.at[pl.ds(s * SUB, SUB)],
                    sem,
                )

        def wait_gather(buf, sem):
            for s in range(SUBS_PER_CHUNK):
                pltpu.make_async_copy(
                    table_hbm.at[idx_all.at[s]],
                    buf.at[pl.ds(s * SUB, SUB)],
                    sem,
                ).wait()

        def scatter_chunk(c, buf):
            for s in range(SUBS_PER_CHUNK):
                pltpu.async_copy(
                    buf.at[pl.ds(s * SUB, SUB)],
                    acc.at[dest_all.at[c * SUBS_PER_CHUNK + s]],
                    ssem,
                    add=True,
                ).wait()

        def process(c, buf, sem, nxt_buf, nxt_sem):
            @pl.when(c + 1 < N_CHUNKS)
            def _():
                start_gather(c + 1, nxt_buf, nxt_sem)
            wait_gather(buf, sem)
            scatter_chunk(c, buf)

        start_gather(0, rows0, gsem0)

        def body(i, carry):
            c = i * 2
            process(c, rows0, gsem0, rows1, gsem1)
            process(c + 1, rows1, gsem1, rows0, gsem0)
            return carry

        lax.fori_loop(0, N_CHUNKS // 2, body, 0)

        pltpu.async_copy(
            acc.at[dest_all.at[IDX_ROWS_PER_W]],
            rows0.at[pl.ds(0, SUB)],
            gsem0,
        ).wait()
        pltpu.sync_copy(rows0.at[pl.ds(0, ROWS_PER_W)],
                        out_hbm.at[pl.ds(wid * ROWS_PER_W, ROWS_PER_W)])

    return k(ids2, dest, zeros, table2)


def _norm_body(s_ref, o_ref):
    x = s_ref[...] * (1.0 / SEQ)
    ss = jnp.sum(x * x, axis=1, keepdims=True)
    n = jnp.sqrt(ss)
    o_ref[...] = x / jnp.maximum(n, 1e-12)


def _normalize(sums):
    return pl.pallas_call(
        _norm_body,
        out_shape=jax.ShapeDtypeStruct((BATCH, DIM), jnp.float32),
    )(sums)


def kernel(input_ids, table):
    table2 = jnp.pad(table, ((0, 0), (0, PDIM - DIM))).reshape(VOCAB2, DIM)
    ids2 = (input_ids.astype(jnp.int32) * 2).reshape(-1, SUB)
    base = (jnp.arange(IDX_PER_W, dtype=jnp.int32) // SEQ).reshape(
        IDX_ROWS_PER_W, SUB)
    ident = jnp.arange(SUB, dtype=jnp.int32)[None, :]
    dest = jnp.concatenate([base, ident], axis=0)
    zeros = jnp.zeros((SUB, DIM), jnp.float32)
    sums = _sc_pool(ids2, dest, zeros, table2)
    return _normalize(sums)
